# baseline (speedup 1.0000x reference)
import jax
import jax.numpy as jnp
from jax.experimental import pallas as pl
from jax.experimental.pallas import tpu as pltpu

_B = 16384
_BLK = 512
_K = 55
_OUT = 220


def _fold_kernel(W1T, b1r, W2T, b2r, W3T, b3r, R1T, c1r, R2T, c2r, R3T, c3r,
                 M_out, a_out, C_out, d_out):
    T12 = jnp.dot(W1T[...], W2T[...], preferred_element_type=jnp.float32)
    M_out[...] = jnp.dot(T12, W3T[...], preferred_element_type=jnp.float32)
    arow = jnp.dot(b1r[...], W2T[...], preferred_element_type=jnp.float32) + b2r[...]
    a_out[...] = jnp.dot(arow, W3T[...], preferred_element_type=jnp.float32) + b3r[...]
    crow = jnp.dot(R1T[...], R2T[...], preferred_element_type=jnp.float32)
    C_out[...] = jnp.dot(crow, R3T[...], preferred_element_type=jnp.float32)
    drow = jnp.dot(c1r[...], R2T[...], preferred_element_type=jnp.float32) + c2r[...]
    d_out[...] = jnp.dot(drow, R3T[...], preferred_element_type=jnp.float32) + c3r[...]


def _main_kernel(xf_ref, e_ref, M_ref, a_ref, C_ref, d_ref,
                 mean_ref, stde_ref, prod_ref):
    z = jnp.dot(xf_ref[...], M_ref[...], preferred_element_type=jnp.float32)
    z = z + a_ref[...]
    mean = z[:, 0:1]
    std = z[:, 1:2]
    sampled = mean + std * e_ref[...]
    mean_ref[...] = mean
    stde_ref[...] = jnp.exp(std)
    prod_ref[...] = sampled * C_ref[...] + d_ref[...]


def kernel(x, W1, b1, W2, b2, W3, b3, R1, c1, R2, c2, R3, c3, e):
    xf = x.reshape(_B, _K)
    f32 = jnp.float32
    M, a, C, d = pl.pallas_call(
        _fold_kernel,
        out_shape=(
            jax.ShapeDtypeStruct((_K, 2), f32),
            jax.ShapeDtypeStruct((1, 2), f32),
            jax.ShapeDtypeStruct((1, _OUT), f32),
            jax.ShapeDtypeStruct((1, _OUT), f32),
        ),
    )(W1.T, b1[None, :], W2.T, b2[None, :], W3.T, b3[None, :],
      R1.reshape(1, 128), c1[None, :], R2.T, c2[None, :], R3.T, c3[None, :])

    grid = (_B // _BLK,)
    mean, stde, prod = pl.pallas_call(
        _main_kernel,
        grid=grid,
        in_specs=[
            pl.BlockSpec((_BLK, _K), lambda i: (i, 0)),
            pl.BlockSpec((_BLK, 1), lambda i: (i, 0)),
            pl.BlockSpec((_K, 2), lambda i: (0, 0)),
            pl.BlockSpec((1, 2), lambda i: (0, 0)),
            pl.BlockSpec((1, _OUT), lambda i: (0, 0)),
            pl.BlockSpec((1, _OUT), lambda i: (0, 0)),
        ],
        out_specs=(
            pl.BlockSpec((_BLK, 1), lambda i: (i, 0)),
            pl.BlockSpec((_BLK, 1), lambda i: (i, 0)),
            pl.BlockSpec((_BLK, _OUT), lambda i: (i, 0)),
        ),
        out_shape=(
            jax.ShapeDtypeStruct((_B, 1), f32),
            jax.ShapeDtypeStruct((_B, 1), f32),
            jax.ShapeDtypeStruct((_B, _OUT), f32),
        ),
    )(xf, e, M, a, C, d)
    return (mean, stde, prod.reshape(_B, 20, 11))
